# Initial kernel scaffold; baseline (speedup 1.0000x reference)
import jax
import jax.numpy as jnp
from jax.experimental import pallas as pl


def kernel(x_scaled, digits, W_enc, W_dec, heads):
    raise NotImplementedError("write your pallas kernel here")


# speedup vs baseline: 1.8151x; 1.8151x over previous
import functools

import jax
import jax.numpy as jnp
from jax.experimental import pallas as pl


def _body(dig_ref, x_ref, enc_ref, dec_ref, heads_ref, rep_ref, recon_ref, z_ref):
    rep = jnp.tanh(jnp.dot(x_ref[...], enc_ref[...], preferred_element_type=jnp.float32))
    rep_ref[...] = rep
    recon_ref[...] = jnp.dot(rep, dec_ref[...], preferred_element_type=jnp.float32)
    dig = dig_ref[...]
    K = heads_ref.shape[0]
    acc = jnp.zeros(z_ref.shape, jnp.float32)
    for k in range(K):
        zk = jnp.dot(rep, heads_ref[k], preferred_element_type=jnp.float32)
        acc = acc + jnp.where(dig == k, zk, 0.0)
    z_ref[...] = acc


def kernel(x_scaled, digits, W_enc, W_dec, heads):
    B, D_IN = x_scaled.shape
    REP = W_enc.shape[1]
    K, _, ZD = heads.shape
    BT = 512
    nb = B // BT
    dig2 = digits.reshape(B, 1)

    rep, recon, z = pl.pallas_call(
        _body,
        grid=(nb,),
        in_specs=[
            pl.BlockSpec((BT, 1), lambda i: (i, 0)),
            pl.BlockSpec((BT, D_IN), lambda i: (i, 0)),
            pl.BlockSpec((D_IN, REP), lambda i: (0, 0)),
            pl.BlockSpec((REP, D_IN), lambda i: (0, 0)),
            pl.BlockSpec((K, REP, ZD), lambda i: (0, 0, 0)),
        ],
        out_specs=[
            pl.BlockSpec((BT, REP), lambda i: (i, 0)),
            pl.BlockSpec((BT, D_IN), lambda i: (i, 0)),
            pl.BlockSpec((BT, ZD), lambda i: (i, 0)),
        ],
        out_shape=[
            jax.ShapeDtypeStruct((B, REP), jnp.float32),
            jax.ShapeDtypeStruct((B, D_IN), jnp.float32),
            jax.ShapeDtypeStruct((B, ZD), jnp.float32),
        ],
    )(dig2, x_scaled, W_enc, W_dec, heads)
    return rep, recon, z
